# baseline (speedup 1.0000x reference)
import functools

import numpy as np
import jax
import jax.numpy as jnp
from jax import lax
from jax.experimental import pallas as pl
from jax.experimental.pallas import tpu as pltpu
from jax.experimental.pallas import tpu_sc as plsc

_NUM_EMB = 1000000
_DIM = 64
_WORDS = 8
_BATCH = 16384
_HIST = 20
_B = _BATCH * _HIST
_ROT_SEED = 42

_PLANE = 1000064
_RPK_BLK = 4096


def _hadamard(d):
    H = np.array([[1.0]], dtype=np.float32)
    while H.shape[0] < d:
        H = np.block([[H, H], [H, -H]]).astype(np.float32)
    return H


def _signs(seed, d):
    rng = np.random.default_rng(seed)
    return rng.integers(0, 2, size=d).astype(np.float32) * 2.0 - 1.0


def _make_rot_const():
    H = _hadamard(_DIM)
    s = _signs(_ROT_SEED, _DIM)
    A = H * s[None, :] / np.float32(_DIM)
    perm = np.array([8 * (j % 8) + j // 8 for j in range(_DIM)])
    return np.ascontiguousarray(A[perm, :])


_A_PERM = _make_rot_const()


def _byte_weights():
    lo = np.zeros((_WORDS, 32), np.float32)
    hi = np.zeros((_WORDS, 32), np.float32)
    for j in range(_WORDS):
        lo[j, 4 * j] = 1.0
        lo[j, 4 * j + 1] = 256.0
        hi[j, 4 * j + 2] = 1.0
        hi[j, 4 * j + 3] = 256.0
    return lo, hi


_W_LO, _W_HI = _byte_weights()


def _repack_body(src_ref, out_ref):
    v = src_ref[...].astype(jnp.int32)
    parts = []
    for j in range(_WORDS):
        b = [lax.slice(v, (4 * j + k, 0), (4 * j + k + 1, v.shape[1]))
             for k in range(4)]
        parts.append(b[0] | (b[1] << 8) | (b[2] << 16) | (b[3] << 24))
    out_ref[...] = jnp.concatenate(parts, axis=0)


def _tc_repack(table_u8_t):
    grid = (_PLANE + _RPK_BLK - 1) // _RPK_BLK
    return pl.pallas_call(
        _repack_body,
        grid=(grid,),
        in_specs=[pl.BlockSpec((32, _RPK_BLK), lambda i: (0, i))],
        out_specs=pl.BlockSpec((_WORDS, _RPK_BLK), lambda i: (0, i)),
        out_shape=jax.ShapeDtypeStruct((_WORDS, _PLANE), jnp.int32),
    )(table_u8_t)


_NW = 32
_BPW = _B // _NW


@functools.cache
def _make_sc_gather():
    def body(ids_hbm, table_hbm, norms_hbm, rows_out, norms_out,
             idx_v, idxc_v, rows_v, nrm_v, sem_r, sem_n):
        wid = lax.axis_index("s") * 2 + lax.axis_index("c")
        base = wid * _BPW
        pltpu.sync_copy(ids_hbm.at[pl.ds(base, _BPW)], idx_v)
        cp_n = pltpu.async_copy(norms_hbm.at[idx_v], nrm_v, sem_n)
        for c in range(_WORDS):
            off = jnp.int32(c * _PLANE)

            def add_off(i, _):
                sl = pl.ds(i * 16, 16)
                idxc_v[sl] = idx_v[sl] + off
                return 0

            lax.fori_loop(0, _BPW // 16, add_off, 0)
            pltpu.async_copy(table_hbm.at[idxc_v], rows_v.at[c], sem_r).wait()
        cp_n.wait()
        pltpu.sync_copy(rows_v, rows_out.at[:, pl.ds(base, _BPW)])
        pltpu.sync_copy(nrm_v, norms_out.at[pl.ds(base, _BPW)])

    return pl.kernel(
        body,
        out_type=[
            jax.ShapeDtypeStruct((_WORDS, _B), jnp.int32),
            jax.ShapeDtypeStruct((_B,), jnp.float32),
        ],
        mesh=plsc.VectorSubcoreMesh(core_axis_name="c", subcore_axis_name="s"),
        compiler_params=pltpu.CompilerParams(use_tc_tiling_on_sc=False),
        scratch_types=[
            pltpu.VMEM((_BPW,), jnp.int32),
            pltpu.VMEM((_BPW,), jnp.int32),
            pltpu.VMEM((_WORDS, _BPW), jnp.int32),
            pltpu.VMEM((_BPW,), jnp.float32),
            pltpu.SemaphoreType.DMA,
            pltpu.SemaphoreType.DMA,
        ],
    )


_BLK = 2048


def _tc_body(rows_ref, norms_ref, cb_ref, rot_ref, out_ref):
    wt = rows_ref[...]
    rows = [(wt >> (4 * k)) & 0xF for k in range(8)]
    idx = jnp.concatenate(rows, axis=0)
    cb2d = jnp.broadcast_to(cb_ref[...][None, :], (_DIM, 16))
    y = jnp.take_along_axis(cb2d, idx, axis=1)
    w = lax.dot_general(
        rot_ref[...], y, (((0,), (0,)), ((), ())),
        preferred_element_type=jnp.float32,
    )
    out_ref[0] = w * norms_ref[...][None, :]


def _tc_dequant(rows_t, norms, codebook):
    nb = _BATCH // _BLK
    return pl.pallas_call(
        _tc_body,
        grid=(_HIST, nb),
        in_specs=[
            pl.BlockSpec((_WORDS, _BLK), lambda h, i: (0, h * nb + i)),
            pl.BlockSpec((_BLK,), lambda h, i: (h * nb + i,)),
            pl.BlockSpec((16,), lambda h, i: (0,)),
            pl.BlockSpec((_DIM, _DIM), lambda h, i: (0, 0)),
        ],
        out_specs=pl.BlockSpec((1, _DIM, _BLK), lambda h, i: (h, 0, i)),
        out_shape=jax.ShapeDtypeStruct((_HIST, _DIM, _BATCH), jnp.float32),
    )(rows_t, norms, codebook, jnp.asarray(_A_PERM))


def kernel(input_ids, indices_packed, weight_norms, codebook):
    flat_ids = input_ids.astype(jnp.int32).T.reshape(-1)
    table1d = _tc_repack(indices_packed.T).reshape(-1)
    rows_t, norms = _make_sc_gather()(flat_ids, table1d, weight_norms)
    out = _tc_dequant(rows_t, norms, codebook)
    return jnp.transpose(out, (2, 0, 1)).astype(codebook.dtype)


# speedup vs baseline: 1.0891x; 1.0891x over previous
import functools

import numpy as np
import jax
import jax.numpy as jnp
from jax import lax
from jax.experimental import pallas as pl
from jax.experimental.pallas import tpu as pltpu
from jax.experimental.pallas import tpu_sc as plsc

_NUM_EMB = 1000000
_DIM = 64
_WORDS = 8
_BATCH = 16384
_HIST = 20
_B = _BATCH * _HIST
_ROT_SEED = 42

_PLANE = 1000064
_RPK_BLK = 16384


def _hadamard(d):
    H = np.array([[1.0]], dtype=np.float32)
    while H.shape[0] < d:
        H = np.block([[H, H], [H, -H]]).astype(np.float32)
    return H


def _signs(seed, d):
    rng = np.random.default_rng(seed)
    return rng.integers(0, 2, size=d).astype(np.float32) * 2.0 - 1.0


def _make_rot_const():
    H = _hadamard(_DIM)
    s = _signs(_ROT_SEED, _DIM)
    A = H * s[None, :] / np.float32(_DIM)
    perm = np.array([8 * (j % 8) + j // 8 for j in range(_DIM)])
    return np.ascontiguousarray(A[perm, :])


_A_PERM = _make_rot_const()


def _byte_weights():
    lo = np.zeros((_WORDS, 32), np.float32)
    hi = np.zeros((_WORDS, 32), np.float32)
    for j in range(_WORDS):
        lo[j, 4 * j] = 1.0
        lo[j, 4 * j + 1] = 256.0
        hi[j, 4 * j + 2] = 1.0
        hi[j, 4 * j + 3] = 256.0
    return lo, hi


_W_LO, _W_HI = _byte_weights()


def _repack_body(src_ref, out_ref):
    v = src_ref[...].astype(jnp.int32)
    parts = []
    for j in range(_WORDS):
        b = [lax.slice(v, (4 * j + k, 0), (4 * j + k + 1, v.shape[1]))
             for k in range(4)]
        parts.append(b[0] | (b[1] << 8) | (b[2] << 16) | (b[3] << 24))
    out_ref[...] = jnp.concatenate(parts, axis=0)


def _tc_repack(table_u8_t):
    grid = (_PLANE + _RPK_BLK - 1) // _RPK_BLK
    return pl.pallas_call(
        _repack_body,
        grid=(grid,),
        in_specs=[pl.BlockSpec((32, _RPK_BLK), lambda i: (0, i))],
        out_specs=pl.BlockSpec((_WORDS, _RPK_BLK), lambda i: (0, i)),
        out_shape=jax.ShapeDtypeStruct((_WORDS, _PLANE), jnp.int32),
    )(table_u8_t)


_NW = 32
_BPW = _B // _NW


@functools.cache
def _make_sc_gather():
    def body(ids_hbm, table_hbm, norms_hbm, rows_out, norms_out,
             idx_v, idxc_v, rows_v, nrm_v, sem_r, sem_n):
        wid = lax.axis_index("s") * 2 + lax.axis_index("c")
        base = wid * _BPW
        pltpu.sync_copy(ids_hbm.at[pl.ds(base, _BPW)], idx_v)
        cp_n = pltpu.async_copy(norms_hbm.at[idx_v], nrm_v, sem_n)
        for c in range(_WORDS):
            off = jnp.int32(c * _PLANE)

            def add_off(i, _):
                sl = pl.ds(i * 16, 16)
                idxc_v[sl] = idx_v[sl] + off
                return 0

            lax.fori_loop(0, _BPW // 16, add_off, 0)
            pltpu.async_copy(table_hbm.at[idxc_v], rows_v.at[c], sem_r).wait()
        cp_n.wait()
        pltpu.sync_copy(rows_v, rows_out.at[:, pl.ds(base, _BPW)])
        pltpu.sync_copy(nrm_v, norms_out.at[pl.ds(base, _BPW)])

    return pl.kernel(
        body,
        out_type=[
            jax.ShapeDtypeStruct((_WORDS, _B), jnp.int32),
            jax.ShapeDtypeStruct((_B,), jnp.float32),
        ],
        mesh=plsc.VectorSubcoreMesh(core_axis_name="c", subcore_axis_name="s"),
        compiler_params=pltpu.CompilerParams(use_tc_tiling_on_sc=False),
        scratch_types=[
            pltpu.VMEM((_BPW,), jnp.int32),
            pltpu.VMEM((_BPW,), jnp.int32),
            pltpu.VMEM((_WORDS, _BPW), jnp.int32),
            pltpu.VMEM((_BPW,), jnp.float32),
            pltpu.SemaphoreType.DMA,
            pltpu.SemaphoreType.DMA,
        ],
    )


_BLK = 2048


def _tc_body(rows_ref, norms_ref, cb_ref, rot_ref, out_ref):
    wt = rows_ref[...]
    rows = [(wt >> (4 * k)) & 0xF for k in range(8)]
    idx = jnp.concatenate(rows, axis=0)
    cb2d = jnp.broadcast_to(cb_ref[...][None, :], (_DIM, 16))
    y = jnp.take_along_axis(cb2d, idx, axis=1)
    w = lax.dot_general(
        rot_ref[...], y, (((0,), (0,)), ((), ())),
        preferred_element_type=jnp.float32,
    )
    out_ref[0] = w * norms_ref[...][None, :]


def _tc_dequant(rows_t, norms, codebook):
    nb = _BATCH // _BLK
    return pl.pallas_call(
        _tc_body,
        grid=(_HIST, nb),
        in_specs=[
            pl.BlockSpec((_WORDS, _BLK), lambda h, i: (0, h * nb + i)),
            pl.BlockSpec((_BLK,), lambda h, i: (h * nb + i,)),
            pl.BlockSpec((16,), lambda h, i: (0,)),
            pl.BlockSpec((_DIM, _DIM), lambda h, i: (0, 0)),
        ],
        out_specs=pl.BlockSpec((1, _DIM, _BLK), lambda h, i: (h, 0, i)),
        out_shape=jax.ShapeDtypeStruct((_HIST, _DIM, _BATCH), jnp.float32),
    )(rows_t, norms, codebook, jnp.asarray(_A_PERM))


def kernel(input_ids, indices_packed, weight_norms, codebook):
    flat_ids = input_ids.astype(jnp.int32).T.reshape(-1)
    table1d = _tc_repack(indices_packed.T).reshape(-1)
    rows_t, norms = _make_sc_gather()(flat_ids, table1d, weight_norms)
    out = _tc_dequant(rows_t, norms, codebook)
    return jnp.transpose(out, (2, 0, 1)).astype(codebook.dtype)
